# baseline (speedup 1.0000x reference)
import functools

import jax
import jax.numpy as jnp
from jax import lax
from jax.experimental import pallas as pl
from jax.experimental.pallas import tpu as pltpu
from jax.experimental.pallas import tpu_sc as plsc

T = 1966080
HOP = 160
FLEN = 400
NF = 1 + (T - FLEN) // HOP
NC, NS, L = 2, 16, 16
NW = NC * NS
FPT = 12288 // NW
SPT = FPT * HOP
HALF = SPT // 2
HALO = (FPT - 1) * HOP + FLEN
NPAIR = 386
WIN = NPAIR * HOP + L
NVEC = WIN // L
FROW = 12288
PSTRIDE = 48

_LN2 = 0.6931471805599453
_SQRT2 = 1.4142135
_INV_TEMP = float(1.0 / (0.1 + 1e-8))
_SIGN_MASK = -2147483648


def _tanh10(x):
    e = jnp.exp(jnp.abs(x) * -20.0)
    r = (1.0 - e) / (1.0 + e)
    xb = lax.bitcast_convert_type(x, jnp.int32)
    rb = lax.bitcast_convert_type(r, jnp.int32)
    return lax.bitcast_convert_type((xb & _SIGN_MASK) | rb, jnp.float32)


def _log(x):
    bits = lax.bitcast_convert_type(x, jnp.int32)
    ex = (bits >> 23) - 127
    m = lax.bitcast_convert_type(
        (bits & 0x007FFFFF) | 0x3F800000, jnp.float32)
    big = m >= _SQRT2
    m = jnp.where(big, m * 0.5, m)
    ef = (ex + big.astype(jnp.int32)).astype(jnp.float32)
    s = (m - 1.0) / (m + 1.0)
    z = s * s
    p = (2.0 * s) * (1.0 + z * (0.33333334 + z * (0.2 + z * 0.14285715)))
    return ef * _LN2 + p


def _sigmoid(x):
    return 1.0 / (1.0 + jnp.exp(-x))


def _sqrt16(v):
    bits = lax.bitcast_convert_type(v, jnp.int32)
    y = lax.bitcast_convert_type((bits >> 1) + 0x1FBD1DF5, jnp.float32)
    for _ in range(3):
        y = 0.5 * (y + v / y)
    return y


def _features_body(x_hbm, feats_hbm, parts_hbm, xb, tb, cbuf, fstage,
                   pstage, sem0, sem1):
    wid = lax.axis_index("s") * NC + lax.axis_index("c")
    s0 = wid * SPT
    c0 = pltpu.async_copy(x_hbm.at[pl.ds(s0, HALF)], xb.at[pl.ds(0, HALF)],
                          sem0)
    c1 = pltpu.async_copy(x_hbm.at[pl.ds(s0 + HALF, HALF)],
                          xb.at[pl.ds(HALF, HALF)], sem1)

    @pl.when(wid < NW - 1)
    def _():
        pltpu.sync_copy(x_hbm.at[pl.ds(s0 + SPT, HALO - SPT)],
                        xb.at[pl.ds(SPT, HALO - SPT)])

    fill_from = jnp.where(wid < NW - 1, HALO // L, SPT // L)

    def _fill(i, c):
        xb[pl.ds(i * L, L)] = jnp.zeros((L,), jnp.float32)
        return c

    lax.fori_loop(fill_from, NVEC, _fill, 0)

    c0.wait()

    @plsc.parallel_loop(0, HALF // L, unroll=8)
    def _tanh_a(v):
        o = v * L
        tb[pl.ds(o, L)] = _tanh10(xb[pl.ds(o, L)])

    c1.wait()

    @plsc.parallel_loop(HALF // L, NVEC, unroll=8)
    def _tanh_b(v):
        o = v * L
        tb[pl.ds(o, L)] = _tanh10(xb[pl.ds(o, L)])

    lanes = lax.iota(jnp.int32, L)
    lane0 = lanes == 0

    def _tree(terms):
        while len(terms) > 1:
            nxt = [terms[i] + terms[i + 1] for i in range(0, len(terms) - 1, 2)]
            if len(terms) % 2:
                nxt.append(terms[-1])
            terms = nxt
        return terms[0]

    @plsc.parallel_loop(0, NPAIR, unroll=2)
    def _pair(p):
        for par in range(2):
            base = p * HOP + par * 80
            t2, ta, td, ts_ = [], [], [], []
            for j in range(5):
                o = base + L * j
                x = xb[pl.ds(o, L)]
                xs = xb[pl.ds(o + 1, L)]
                t = tb[pl.ds(o, L)]
                ts = tb[pl.ds(o + 1, L)]
                t2.append(x * x)
                ta.append(jnp.abs(x))
                d = xs - x
                td.append(d * d)
                ts_.append(jnp.abs(ts - t))
            a2 = _tree(t2)
            aa = _tree(ta)
            ad = _tree(td)
            as_ = _tree(ts_)
            for k, acc in ((0, a2), (1, as_), (2, aa), (3, ad)):
                dest = jnp.full((L,), (k * 2 + par) * 400 + p, jnp.int32)
                plsc.store_scatter(cbuf, [dest],
                                   jnp.full((L,), jnp.sum(acc), jnp.float32),
                                   mask=lane0)

    @plsc.parallel_loop(0, FPT // L, unroll=2)
    def _frames(vb):
        fl = vb * L + lanes
        idx = fl * HOP + (FLEN - 1)
        xa = plsc.load_gather(xb, [idx])
        xz = plsc.load_gather(xb, [idx + 1])
        d2c = (xz - xa) * (xz - xa)
        dsc = jnp.abs(plsc.load_gather(tb, [idx + 1])
                      - plsc.load_gather(tb, [idx]))

        def fsum(k):
            ce = (k * 2 + 0) * 400
            co = (k * 2 + 1) * 400
            o = vb * L
            return (cbuf[pl.ds(ce + o, L)] + cbuf[pl.ds(co + o, L)]
                    + cbuf[pl.ds(ce + o + 1, L)] + cbuf[pl.ds(co + o + 1, L)]
                    + cbuf[pl.ds(ce + o + 2, L)])

        energy = _log(fsum(0) * (1.0 / FLEN) + 1e-8)
        zcr = (fsum(1) - dsc) * (0.5 / (FLEN - 1))
        amp = fsum(2) * (1.0 / FLEN)
        hf = _log((fsum(3) - d2c) * (1.0 / (FLEN - 1)) + 1e-8)
        fstage[pl.ds(0 * FPT + vb * L, L)] = energy
        fstage[pl.ds(1 * FPT + vb * L, L)] = zcr
        fstage[pl.ds(2 * FPT + vb * L, L)] = amp
        fstage[pl.ds(3 * FPT + vb * L, L)] = hf

    fb = wid * FPT
    nval_i = jnp.maximum(jnp.minimum(FPT, NF - fb), 1)
    invn = 1.0 / jnp.full((L,), nval_i.astype(jnp.float32), jnp.float32)
    svec = jnp.zeros((L,), jnp.float32)
    mvec = jnp.zeros((L,), jnp.float32)
    qvec = jnp.zeros((L,), jnp.float32)
    for k in range(4):
        acc = jnp.zeros((L,), jnp.float32)
        for vb in range(FPT // L):
            f = fstage[pl.ds(k * FPT + vb * L, L)]
            valid = (vb * L + lanes) < nval_i
            acc = acc + jnp.where(valid, f, 0.0)
        s1 = jnp.sum(acc)
        mu_i = s1 * invn
        acc2 = jnp.zeros((L,), jnp.float32)
        for vb in range(FPT // L):
            f = fstage[pl.ds(k * FPT + vb * L, L)]
            valid = (vb * L + lanes) < nval_i
            d = jnp.where(valid, f - mu_i, 0.0)
            acc2 = acc2 + d * d
        s2 = jnp.sum(acc2)
        sel = lanes == k
        svec = jnp.where(sel, s1, svec)
        mvec = jnp.where(sel, mu_i, mvec)
        qvec = jnp.where(sel, s2, qvec)
    pstage[pl.ds(0, L)] = svec
    pstage[pl.ds(L, L)] = mvec
    pstage[pl.ds(2 * L, L)] = qvec
    pltpu.sync_copy(pstage, parts_hbm.at[pl.ds(wid * PSTRIDE, PSTRIDE)])
    for k in range(4):
        pltpu.sync_copy(fstage.at[pl.ds(k * FPT, FPT)],
                        feats_hbm.at[pl.ds(k * FROW + fb, FPT)])


def _apply_body(x_hbm, feats_hbm, parts_hbm, params_hbm, out_hbm, xb,
                featsb, partsb, mbuf, pbuf, sem0, sem1, sem2):
    wid = lax.axis_index("s") * NC + lax.axis_index("c")
    s0 = wid * SPT
    xc0 = pltpu.async_copy(x_hbm.at[pl.ds(s0, HALF)], xb.at[pl.ds(0, HALF)],
                           sem0)
    xc1 = pltpu.async_copy(x_hbm.at[pl.ds(s0 + HALF, HALF)],
                           xb.at[pl.ds(HALF, HALF)], sem1)
    pltpu.sync_copy(params_hbm, pbuf)
    pltpu.sync_copy(parts_hbm, partsb)
    fb = wid * FPT
    for k in range(4):
        pltpu.sync_copy(feats_hbm.at[pl.ds(k * FROW + fb, FPT)],
                        featsb.at[pl.ds(k * FPT, FPT)])

    lanes = lax.iota(jnp.int32, L)
    nt0 = jnp.minimum(FPT, NF - FPT * lanes).astype(jnp.float32)
    nt1 = jnp.minimum(FPT, NF - FPT * (lanes + L)).astype(jnp.float32)

    mus = []
    invsd = []
    for k in range(4):
        s1 = (plsc.load_gather(partsb, [lanes * PSTRIDE + k])
              + plsc.load_gather(partsb, [(lanes + L) * PSTRIDE + k]))
        mu = jnp.sum(s1) * (1.0 / NF)
        m0 = plsc.load_gather(partsb, [lanes * PSTRIDE + L + k])
        m1 = plsc.load_gather(partsb, [(lanes + L) * PSTRIDE + L + k])
        q0 = plsc.load_gather(partsb, [lanes * PSTRIDE + 2 * L + k])
        q1 = plsc.load_gather(partsb, [(lanes + L) * PSTRIDE + 2 * L + k])
        d0 = m0 - mu
        d1 = m1 - mu
        var = jnp.sum(q0 + q1 + nt0 * d0 * d0 + nt1 * d1 * d1) * (1.0 / NF)
        sd = _sqrt16(jnp.full((L,), var, jnp.float32))
        mus.append(mu)
        invsd.append(1.0 / (sd + 1e-6))

    pv = pbuf[...]
    e = jnp.exp(jnp.where(lanes < 4, pv, -1e30))
    wsm = e / jnp.full((L,), jnp.sum(e), jnp.float32)
    wk = [jnp.sum(jnp.where(lanes == k, wsm, 0.0)) for k in range(4)]
    thr = jnp.sum(jnp.where(lanes == 4, pv, 0.0))

    @plsc.parallel_loop(0, FPT // L, unroll=2)
    def _score(vb):
        fg = fb + vb * L + lanes
        lin = jnp.zeros((L,), jnp.float32)
        for k in range(4):
            f = featsb[pl.ds(k * FPT + vb * L, L)]
            lin = lin + wk[k] * ((f - mus[k]) * invsd[k])
        score = _sigmoid(lin)
        score = jnp.where(fg < NF, score, 0.0)
        mbuf[pl.ds(vb * L, L)] = _sigmoid((score - thr) * _INV_TEMP)

    xc0.wait()

    @plsc.parallel_loop(0, FPT // 2, unroll=2)
    def _apply_a(fl):
        mv = plsc.load_gather(mbuf, [jnp.full((L,), fl, jnp.int32)])
        g = 1.0 - mv
        for j in range(HOP // L):
            o = fl * HOP + j * L
            xb[pl.ds(o, L)] = xb[pl.ds(o, L)] * g

    oc0 = pltpu.async_copy(xb.at[pl.ds(0, HALF)], out_hbm.at[pl.ds(s0, HALF)],
                           sem2)
    xc1.wait()

    @plsc.parallel_loop(FPT // 2, FPT, unroll=2)
    def _apply_b(fl):
        mv = plsc.load_gather(mbuf, [jnp.full((L,), fl, jnp.int32)])
        g = 1.0 - mv
        for j in range(HOP // L):
            o = fl * HOP + j * L
            xb[pl.ds(o, L)] = xb[pl.ds(o, L)] * g

    pltpu.sync_copy(xb.at[pl.ds(HALF, HALF)],
                    out_hbm.at[pl.ds(s0 + HALF, HALF)])
    oc0.wait()


_sc_mesh = plsc.VectorSubcoreMesh(core_axis_name="c", subcore_axis_name="s")

_features = functools.partial(
    pl.kernel,
    out_type=(
        jax.ShapeDtypeStruct((4 * FROW,), jnp.float32),
        jax.ShapeDtypeStruct((NW * PSTRIDE,), jnp.float32),
    ),
    mesh=_sc_mesh,
    compiler_params=pltpu.CompilerParams(needs_layout_passes=False),
    scratch_types=[
        pltpu.VMEM((WIN,), jnp.float32),
        pltpu.VMEM((WIN,), jnp.float32),
        pltpu.VMEM((8 * 400,), jnp.float32),
        pltpu.VMEM((4 * FPT,), jnp.float32),
        pltpu.VMEM((PSTRIDE,), jnp.float32),
        pltpu.SemaphoreType.DMA,
        pltpu.SemaphoreType.DMA,
    ],
)(_features_body)

_apply = functools.partial(
    pl.kernel,
    out_type=jax.ShapeDtypeStruct((T,), jnp.float32),
    mesh=_sc_mesh,
    compiler_params=pltpu.CompilerParams(needs_layout_passes=False),
    scratch_types=[
        pltpu.VMEM((SPT,), jnp.float32),
        pltpu.VMEM((4 * FPT,), jnp.float32),
        pltpu.VMEM((NW * PSTRIDE,), jnp.float32),
        pltpu.VMEM((FPT,), jnp.float32),
        pltpu.VMEM((L,), jnp.float32),
        pltpu.SemaphoreType.DMA,
        pltpu.SemaphoreType.DMA,
        pltpu.SemaphoreType.DMA,
    ],
)(_apply_body)


def kernel(waveform_tensor, weight_logits, soft_threshold):
    wf = waveform_tensor
    if wf.ndim > 1:
        wf = wf.mean(axis=0)
    wf = wf.astype(jnp.float32)
    params = jnp.zeros((L,), jnp.float32)
    params = params.at[0:4].set(weight_logits.astype(jnp.float32))
    params = params.at[4].set(jnp.asarray(soft_threshold, jnp.float32))
    feats, parts = _features(wf)
    return _apply(wf, feats, parts, params)


# speedup vs baseline: 1.0179x; 1.0179x over previous
import functools

import jax
import jax.numpy as jnp
from jax import lax
from jax.experimental import pallas as pl
from jax.experimental.pallas import tpu as pltpu
from jax.experimental.pallas import tpu_sc as plsc

T = 1966080
HOP = 160
FLEN = 400
NF = 1 + (T - FLEN) // HOP
NC, NS, L = 2, 16, 16
NW = NC * NS
FPT = 12288 // NW
SPT = FPT * HOP
HALF = SPT // 2
HALO = (FPT - 1) * HOP + FLEN
NPAIR = 386
WIN = NPAIR * HOP + L
NVEC = WIN // L
FROW = 12288
PSTRIDE = 48

_LN2 = 0.6931471805599453
_SQRT2 = 1.4142135
_INV_TEMP = float(1.0 / (0.1 + 1e-8))
_SIGN_MASK = -2147483648


def _tanh10(x):
    e = jnp.exp(jnp.abs(x) * -20.0)
    r = (1.0 - e) / (1.0 + e)
    xb = lax.bitcast_convert_type(x, jnp.int32)
    rb = lax.bitcast_convert_type(r, jnp.int32)
    return lax.bitcast_convert_type((xb & _SIGN_MASK) | rb, jnp.float32)


def _log(x):
    bits = lax.bitcast_convert_type(x, jnp.int32)
    ex = (bits >> 23) - 127
    m = lax.bitcast_convert_type(
        (bits & 0x007FFFFF) | 0x3F800000, jnp.float32)
    big = m >= _SQRT2
    m = jnp.where(big, m * 0.5, m)
    ef = (ex + big.astype(jnp.int32)).astype(jnp.float32)
    s = (m - 1.0) / (m + 1.0)
    z = s * s
    p = (2.0 * s) * (1.0 + z * (0.33333334 + z * (0.2 + z * 0.14285715)))
    return ef * _LN2 + p


def _sigmoid(x):
    return 1.0 / (1.0 + jnp.exp(-x))


def _sqrt16(v):
    bits = lax.bitcast_convert_type(v, jnp.int32)
    y = lax.bitcast_convert_type((bits >> 1) + 0x1FBD1DF5, jnp.float32)
    for _ in range(3):
        y = 0.5 * (y + v / y)
    return y


def _features_body(x_hbm, feats_hbm, parts_hbm, xb, tb, cbuf, fstage,
                   pstage, sem0, sem1):
    wid = lax.axis_index("s") * NC + lax.axis_index("c")
    s0 = wid * SPT
    c0 = pltpu.async_copy(x_hbm.at[pl.ds(s0, HALF)], xb.at[pl.ds(0, HALF)],
                          sem0)
    c1 = pltpu.async_copy(x_hbm.at[pl.ds(s0 + HALF, HALF)],
                          xb.at[pl.ds(HALF, HALF)], sem1)

    @pl.when(wid < NW - 1)
    def _():
        pltpu.sync_copy(x_hbm.at[pl.ds(s0 + SPT, HALO - SPT)],
                        xb.at[pl.ds(SPT, HALO - SPT)])

    fill_from = jnp.where(wid < NW - 1, HALO // L, SPT // L)

    def _fill(i, c):
        xb[pl.ds(i * L, L)] = jnp.zeros((L,), jnp.float32)
        return c

    lax.fori_loop(fill_from, NVEC, _fill, 0)

    c0.wait()

    @plsc.parallel_loop(0, HALF // L, unroll=8)
    def _tanh_a(v):
        o = v * L
        tb[pl.ds(o, L)] = _tanh10(xb[pl.ds(o, L)])

    c1.wait()

    @plsc.parallel_loop(HALF // L, NVEC, unroll=8)
    def _tanh_b(v):
        o = v * L
        tb[pl.ds(o, L)] = _tanh10(xb[pl.ds(o, L)])

    lanes = lax.iota(jnp.int32, L)
    lane0 = lanes == 0

    @plsc.parallel_loop(0, NPAIR, unroll=1)
    def _pair(p):
        for par in range(2):
            base = p * HOP + par * 80
            a2 = jnp.zeros((L,), jnp.float32)
            aa = jnp.zeros((L,), jnp.float32)
            ad = jnp.zeros((L,), jnp.float32)
            as_ = jnp.zeros((L,), jnp.float32)
            for j in range(5):
                o = base + L * j
                x = xb[pl.ds(o, L)]
                xs = xb[pl.ds(o + 1, L)]
                t = tb[pl.ds(o, L)]
                ts = tb[pl.ds(o + 1, L)]
                a2 = a2 + x * x
                aa = aa + jnp.abs(x)
                d = xs - x
                ad = ad + d * d
                as_ = as_ + jnp.abs(ts - t)
            for k, acc in ((0, a2), (1, as_), (2, aa), (3, ad)):
                dest = jnp.full((L,), (k * 2 + par) * 400 + p, jnp.int32)
                plsc.store_scatter(cbuf, [dest],
                                   jnp.full((L,), jnp.sum(acc), jnp.float32),
                                   mask=lane0)

    @plsc.parallel_loop(0, FPT // L, unroll=2)
    def _frames(vb):
        fl = vb * L + lanes
        idx = fl * HOP + (FLEN - 1)
        xa = plsc.load_gather(xb, [idx])
        xz = plsc.load_gather(xb, [idx + 1])
        d2c = (xz - xa) * (xz - xa)
        dsc = jnp.abs(plsc.load_gather(tb, [idx + 1])
                      - plsc.load_gather(tb, [idx]))

        def fsum(k):
            ce = (k * 2 + 0) * 400
            co = (k * 2 + 1) * 400
            o = vb * L
            return (cbuf[pl.ds(ce + o, L)] + cbuf[pl.ds(co + o, L)]
                    + cbuf[pl.ds(ce + o + 1, L)] + cbuf[pl.ds(co + o + 1, L)]
                    + cbuf[pl.ds(ce + o + 2, L)])

        energy = _log(fsum(0) * (1.0 / FLEN) + 1e-8)
        zcr = (fsum(1) - dsc) * (0.5 / (FLEN - 1))
        amp = fsum(2) * (1.0 / FLEN)
        hf = _log((fsum(3) - d2c) * (1.0 / (FLEN - 1)) + 1e-8)
        fstage[pl.ds(0 * FPT + vb * L, L)] = energy
        fstage[pl.ds(1 * FPT + vb * L, L)] = zcr
        fstage[pl.ds(2 * FPT + vb * L, L)] = amp
        fstage[pl.ds(3 * FPT + vb * L, L)] = hf

    fb = wid * FPT
    nval_i = jnp.maximum(jnp.minimum(FPT, NF - fb), 1)
    invn = 1.0 / jnp.full((L,), nval_i.astype(jnp.float32), jnp.float32)
    svec = jnp.zeros((L,), jnp.float32)
    mvec = jnp.zeros((L,), jnp.float32)
    qvec = jnp.zeros((L,), jnp.float32)
    for k in range(4):
        acc = jnp.zeros((L,), jnp.float32)
        for vb in range(FPT // L):
            f = fstage[pl.ds(k * FPT + vb * L, L)]
            valid = (vb * L + lanes) < nval_i
            acc = acc + jnp.where(valid, f, 0.0)
        s1 = jnp.sum(acc)
        mu_i = s1 * invn
        acc2 = jnp.zeros((L,), jnp.float32)
        for vb in range(FPT // L):
            f = fstage[pl.ds(k * FPT + vb * L, L)]
            valid = (vb * L + lanes) < nval_i
            d = jnp.where(valid, f - mu_i, 0.0)
            acc2 = acc2 + d * d
        s2 = jnp.sum(acc2)
        sel = lanes == k
        svec = jnp.where(sel, s1, svec)
        mvec = jnp.where(sel, mu_i, mvec)
        qvec = jnp.where(sel, s2, qvec)
    pstage[pl.ds(0, L)] = svec
    pstage[pl.ds(L, L)] = mvec
    pstage[pl.ds(2 * L, L)] = qvec
    pltpu.sync_copy(pstage, parts_hbm.at[pl.ds(wid * PSTRIDE, PSTRIDE)])
    for k in range(4):
        pltpu.sync_copy(fstage.at[pl.ds(k * FPT, FPT)],
                        feats_hbm.at[pl.ds(k * FROW + fb, FPT)])


def _apply_body(x_hbm, feats_hbm, parts_hbm, params_hbm, out_hbm, xb,
                featsb, partsb, mbuf, pbuf, sem0, sem1, sem2):
    wid = lax.axis_index("s") * NC + lax.axis_index("c")
    s0 = wid * SPT
    xc0 = pltpu.async_copy(x_hbm.at[pl.ds(s0, HALF)], xb.at[pl.ds(0, HALF)],
                           sem0)
    xc1 = pltpu.async_copy(x_hbm.at[pl.ds(s0 + HALF, HALF)],
                           xb.at[pl.ds(HALF, HALF)], sem1)
    pltpu.sync_copy(params_hbm, pbuf)
    pltpu.sync_copy(parts_hbm, partsb)
    fb = wid * FPT
    for k in range(4):
        pltpu.sync_copy(feats_hbm.at[pl.ds(k * FROW + fb, FPT)],
                        featsb.at[pl.ds(k * FPT, FPT)])

    lanes = lax.iota(jnp.int32, L)
    nt0 = jnp.minimum(FPT, NF - FPT * lanes).astype(jnp.float32)
    nt1 = jnp.minimum(FPT, NF - FPT * (lanes + L)).astype(jnp.float32)

    mus = []
    invsd = []
    for k in range(4):
        s1 = (plsc.load_gather(partsb, [lanes * PSTRIDE + k])
              + plsc.load_gather(partsb, [(lanes + L) * PSTRIDE + k]))
        mu = jnp.sum(s1) * (1.0 / NF)
        m0 = plsc.load_gather(partsb, [lanes * PSTRIDE + L + k])
        m1 = plsc.load_gather(partsb, [(lanes + L) * PSTRIDE + L + k])
        q0 = plsc.load_gather(partsb, [lanes * PSTRIDE + 2 * L + k])
        q1 = plsc.load_gather(partsb, [(lanes + L) * PSTRIDE + 2 * L + k])
        d0 = m0 - mu
        d1 = m1 - mu
        var = jnp.sum(q0 + q1 + nt0 * d0 * d0 + nt1 * d1 * d1) * (1.0 / NF)
        sd = _sqrt16(jnp.full((L,), var, jnp.float32))
        mus.append(mu)
        invsd.append(1.0 / (sd + 1e-6))

    pv = pbuf[...]
    e = jnp.exp(jnp.where(lanes < 4, pv, -1e30))
    wsm = e / jnp.full((L,), jnp.sum(e), jnp.float32)
    wk = [jnp.sum(jnp.where(lanes == k, wsm, 0.0)) for k in range(4)]
    thr = jnp.sum(jnp.where(lanes == 4, pv, 0.0))

    @plsc.parallel_loop(0, FPT // L, unroll=2)
    def _score(vb):
        fg = fb + vb * L + lanes
        lin = jnp.zeros((L,), jnp.float32)
        for k in range(4):
            f = featsb[pl.ds(k * FPT + vb * L, L)]
            lin = lin + wk[k] * ((f - mus[k]) * invsd[k])
        score = _sigmoid(lin)
        score = jnp.where(fg < NF, score, 0.0)
        mbuf[pl.ds(vb * L, L)] = _sigmoid((score - thr) * _INV_TEMP)

    xc0.wait()

    @plsc.parallel_loop(0, FPT // 2, unroll=2)
    def _apply_a(fl):
        mv = plsc.load_gather(mbuf, [jnp.full((L,), fl, jnp.int32)])
        g = 1.0 - mv
        for j in range(HOP // L):
            o = fl * HOP + j * L
            xb[pl.ds(o, L)] = xb[pl.ds(o, L)] * g

    oc0 = pltpu.async_copy(xb.at[pl.ds(0, HALF)], out_hbm.at[pl.ds(s0, HALF)],
                           sem2)
    xc1.wait()

    @plsc.parallel_loop(FPT // 2, FPT, unroll=2)
    def _apply_b(fl):
        mv = plsc.load_gather(mbuf, [jnp.full((L,), fl, jnp.int32)])
        g = 1.0 - mv
        for j in range(HOP // L):
            o = fl * HOP + j * L
            xb[pl.ds(o, L)] = xb[pl.ds(o, L)] * g

    pltpu.sync_copy(xb.at[pl.ds(HALF, HALF)],
                    out_hbm.at[pl.ds(s0 + HALF, HALF)])
    oc0.wait()


_sc_mesh = plsc.VectorSubcoreMesh(core_axis_name="c", subcore_axis_name="s")

_features = functools.partial(
    pl.kernel,
    out_type=(
        jax.ShapeDtypeStruct((4 * FROW,), jnp.float32),
        jax.ShapeDtypeStruct((NW * PSTRIDE,), jnp.float32),
    ),
    mesh=_sc_mesh,
    compiler_params=pltpu.CompilerParams(needs_layout_passes=False),
    scratch_types=[
        pltpu.VMEM((WIN,), jnp.float32),
        pltpu.VMEM((WIN,), jnp.float32),
        pltpu.VMEM((8 * 400,), jnp.float32),
        pltpu.VMEM((4 * FPT,), jnp.float32),
        pltpu.VMEM((PSTRIDE,), jnp.float32),
        pltpu.SemaphoreType.DMA,
        pltpu.SemaphoreType.DMA,
    ],
)(_features_body)

_apply = functools.partial(
    pl.kernel,
    out_type=jax.ShapeDtypeStruct((T,), jnp.float32),
    mesh=_sc_mesh,
    compiler_params=pltpu.CompilerParams(needs_layout_passes=False),
    scratch_types=[
        pltpu.VMEM((SPT,), jnp.float32),
        pltpu.VMEM((4 * FPT,), jnp.float32),
        pltpu.VMEM((NW * PSTRIDE,), jnp.float32),
        pltpu.VMEM((FPT,), jnp.float32),
        pltpu.VMEM((L,), jnp.float32),
        pltpu.SemaphoreType.DMA,
        pltpu.SemaphoreType.DMA,
        pltpu.SemaphoreType.DMA,
    ],
)(_apply_body)


def kernel(waveform_tensor, weight_logits, soft_threshold):
    wf = waveform_tensor
    if wf.ndim > 1:
        wf = wf.mean(axis=0)
    wf = wf.astype(jnp.float32)
    params = jnp.zeros((L,), jnp.float32)
    params = params.at[0:4].set(weight_logits.astype(jnp.float32))
    params = params.at[4].set(jnp.asarray(soft_threshold, jnp.float32))
    feats, parts = _features(wf)
    return _apply(wf, feats, parts, params)
